# Initial kernel scaffold; baseline (speedup 1.0000x reference)
import jax
import jax.numpy as jnp
from jax.experimental import pallas as pl


def kernel(x, hour, day_of_week, month, pe, hour_emb, dow_emb, month_emb):
    raise NotImplementedError("write your pallas kernel here")


# speedup vs baseline: 1.3965x; 1.3965x over previous
import functools

import jax
import jax.numpy as jnp
from jax import lax
from jax.experimental import pallas as pl
from jax.experimental.pallas import tpu as pltpu
from jax.experimental.pallas import tpu_sc as plsc

D4 = 256


def _sc_temporal(idx, table, batch):
    n_idx = idx.shape[0]

    mesh = plsc.VectorSubcoreMesh(core_axis_name="c", subcore_axis_name="s")

    @functools.partial(
        pl.kernel,
        mesh=mesh,
        out_type=jax.ShapeDtypeStruct((batch, 4 * D4), jnp.float32),
        scratch_types=[
            pltpu.VMEM((n_idx,), jnp.int32),
            pltpu.VMEM((n_idx, D4), jnp.float32),
            pltpu.VMEM((batch, D4), jnp.float32),
            pltpu.SemaphoreType.DMA,
        ],
    )
    def sc_kernel(idx_hbm, tbl_hbm, out_hbm, idx_v, rows_v, zero_v, sem):
        wid = lax.axis_index("s") * 2 + lax.axis_index("c")

        @pl.when(wid == 0)
        def _():
            pltpu.sync_copy(idx_hbm, idx_v)
            pltpu.async_copy(tbl_hbm.at[idx_v], rows_v, sem).wait()
            pltpu.sync_copy(rows_v.at[pl.ds(0, batch)],
                            out_hbm.at[:, pl.ds(0 * D4, D4)])
            pltpu.sync_copy(rows_v.at[pl.ds(batch, batch)],
                            out_hbm.at[:, pl.ds(1 * D4, D4)])
            pltpu.sync_copy(rows_v.at[pl.ds(2 * batch, batch)],
                            out_hbm.at[:, pl.ds(2 * D4, D4)])
            zvec = jnp.zeros((16,), jnp.float32)
            for j in range(batch):
                for i in range(D4 // 16):
                    zero_v[j, pl.ds(i * 16, 16)] = zvec
            pltpu.sync_copy(zero_v, out_hbm.at[:, pl.ds(3 * D4, D4)])

    return sc_kernel(idx, table)


def _tc_body(x_ref, pe_ref, t_ref, o_ref):
    o_ref[...] = x_ref[...] + pe_ref[...][None, :, :] + t_ref[...][:, None, :]


def _tc_add(x, pe_used, temporal, seq_block):
    batch, seq, d_model = x.shape
    grid = (seq // seq_block,)
    return pl.pallas_call(
        _tc_body,
        grid=grid,
        in_specs=[
            pl.BlockSpec((batch, seq_block, d_model), lambda i: (0, i, 0)),
            pl.BlockSpec((seq_block, d_model), lambda i: (i, 0)),
            pl.BlockSpec((batch, d_model), lambda i: (0, 0)),
        ],
        out_specs=pl.BlockSpec((batch, seq_block, d_model), lambda i: (0, i, 0)),
        out_shape=jax.ShapeDtypeStruct((batch, seq, d_model), jnp.float32),
        compiler_params=pltpu.CompilerParams(
            dimension_semantics=("arbitrary",),
        ),
    )(x, pe_used, temporal)


def kernel(x, hour, day_of_week, month, pe, hour_emb, dow_emb, month_emb):
    batch, seq, d_model = x.shape
    n_hour = hour_emb.shape[0]
    n_dow = dow_emb.shape[0]

    table = jnp.concatenate([hour_emb, dow_emb, month_emb], axis=0)
    idx = jnp.concatenate([
        hour.astype(jnp.int32),
        day_of_week.astype(jnp.int32) + n_hour,
        month.astype(jnp.int32) + n_hour + n_dow,
    ])
    n_idx = 16
    idx = jnp.pad(idx, (0, n_idx - idx.shape[0]))

    temporal = _sc_temporal(idx, table, batch)

    pe_used = pe[:seq, :]
    seq_block = 256
    return _tc_add(x, pe_used, temporal, seq_block)


# speedup vs baseline: 1.4041x; 1.0055x over previous
import functools

import jax
import jax.numpy as jnp
from jax import lax
from jax.experimental import pallas as pl
from jax.experimental.pallas import tpu as pltpu
from jax.experimental.pallas import tpu_sc as plsc

D4 = 256


def _sc_temporal(idx, table, batch):
    n_idx = idx.shape[0]

    mesh = plsc.VectorSubcoreMesh(core_axis_name="c", subcore_axis_name="s")

    @functools.partial(
        pl.kernel,
        mesh=mesh,
        out_type=jax.ShapeDtypeStruct((batch, 4 * D4), jnp.float32),
        scratch_types=[
            pltpu.VMEM((n_idx,), jnp.int32),
            pltpu.VMEM((n_idx, D4), jnp.float32),
            pltpu.VMEM((batch, D4), jnp.float32),
            pltpu.SemaphoreType.DMA,
        ],
    )
    def sc_kernel(idx_hbm, tbl_hbm, out_hbm, idx_v, rows_v, zero_v, sem):
        wid = lax.axis_index("s") * 2 + lax.axis_index("c")

        @pl.when(wid == 0)
        def _():
            pltpu.sync_copy(idx_hbm, idx_v)
            pltpu.async_copy(tbl_hbm.at[idx_v], rows_v, sem).wait()
            pltpu.sync_copy(rows_v.at[pl.ds(0, batch)],
                            out_hbm.at[:, pl.ds(0 * D4, D4)])
            pltpu.sync_copy(rows_v.at[pl.ds(batch, batch)],
                            out_hbm.at[:, pl.ds(1 * D4, D4)])
            pltpu.sync_copy(rows_v.at[pl.ds(2 * batch, batch)],
                            out_hbm.at[:, pl.ds(2 * D4, D4)])
            zvec = jnp.zeros((16,), jnp.float32)
            for j in range(batch):
                for i in range(D4 // 16):
                    zero_v[j, pl.ds(i * 16, 16)] = zvec
            pltpu.sync_copy(zero_v, out_hbm.at[:, pl.ds(3 * D4, D4)])

    return sc_kernel(idx, table)


def _tc_body(x_ref, pe_ref, t_ref, o_ref):
    o_ref[...] = x_ref[...] + pe_ref[...][None, :, :] + t_ref[...][:, None, :]


def _tc_add(x, pe_used, temporal, seq_block):
    batch, seq, d_model = x.shape
    grid = (seq // seq_block,)
    return pl.pallas_call(
        _tc_body,
        grid=grid,
        in_specs=[
            pl.BlockSpec((batch, seq_block, d_model), lambda i: (0, i, 0)),
            pl.BlockSpec((seq_block, d_model), lambda i: (i, 0)),
            pl.BlockSpec((batch, d_model), lambda i: (0, 0)),
        ],
        out_specs=pl.BlockSpec((batch, seq_block, d_model), lambda i: (0, i, 0)),
        out_shape=jax.ShapeDtypeStruct((batch, seq, d_model), jnp.float32),
        compiler_params=pltpu.CompilerParams(
            dimension_semantics=("arbitrary",),
        ),
    )(x, pe_used, temporal)


def kernel(x, hour, day_of_week, month, pe, hour_emb, dow_emb, month_emb):
    batch, seq, d_model = x.shape
    n_hour = hour_emb.shape[0]
    n_dow = dow_emb.shape[0]

    table = jnp.concatenate([hour_emb, dow_emb, month_emb], axis=0)
    idx = jnp.concatenate([
        hour.astype(jnp.int32),
        day_of_week.astype(jnp.int32) + n_hour,
        month.astype(jnp.int32) + n_hour + n_dow,
    ])
    n_idx = 16
    idx = jnp.pad(idx, (0, n_idx - idx.shape[0]))

    temporal = _sc_temporal(idx, table, batch)

    pe_used = pe[:seq, :]
    seq_block = 512
    return _tc_add(x, pe_used, temporal, seq_block)
